# baseline (speedup 1.0000x reference)
import jax
import jax.numpy as jnp
from jax.experimental import pallas as pl
from jax.experimental.pallas import tpu as pltpu

_N = 100000
_K = 384
_M = 128
_BA = 5000
_TA = _N // _BA
_BB = 5000
_TB = _N // _BB


def _bn_linear_kernel(x_ref, w_ref, b_ref, g_ref, beta_ref, out_ref,
                      y_ref, sum_ref, sq_ref):
    i = pl.program_id(0)

    @pl.when(i < _TA)
    def _compute_phase():
        y = jax.lax.dot_general(
            x_ref[...], w_ref[...],
            dimension_numbers=(((1,), (1,)), ((), ())),
            preferred_element_type=jnp.float32,
        ) + b_ref[...]
        y_ref[pl.ds(i * _BA, _BA), :] = y.astype(jnp.bfloat16)
        out_ref[...] = y
        ps = jnp.sum(y, axis=0, keepdims=True)
        pq = jnp.sum(y * y, axis=0, keepdims=True)

        @pl.when(i == 0)
        def _():
            sum_ref[...] = ps
            sq_ref[...] = pq

        @pl.when(i > 0)
        def _():
            sum_ref[...] += ps
            sq_ref[...] += pq

    @pl.when(i >= _TA)
    def _normalize_phase():
        j = i - _TA
        inv_n = 1.0 / _N
        mean = sum_ref[...] * inv_n
        var = sq_ref[...] * inv_n - mean * mean
        scale = g_ref[...] * jax.lax.rsqrt(var + 1e-5)
        shift = beta_ref[...] - mean * scale
        out_ref[...] = y_ref[pl.ds(j * _BB, _BB), :].astype(jnp.float32) * scale + shift


@jax.jit
def kernel(x, W, b, gamma, beta):
    b2 = b.reshape(1, _M)
    g2 = gamma.reshape(1, _M)
    beta2 = beta.reshape(1, _M)
    return pl.pallas_call(
        _bn_linear_kernel,
        grid=(_TA + _TB,),
        in_specs=[
            pl.BlockSpec((_BA, _K), lambda i: (jnp.minimum(i, _TA - 1), 0)),
            pl.BlockSpec((_M, _K), lambda i: (0, 0)),
            pl.BlockSpec((1, _M), lambda i: (0, 0)),
            pl.BlockSpec((1, _M), lambda i: (0, 0)),
            pl.BlockSpec((1, _M), lambda i: (0, 0)),
        ],
        out_specs=pl.BlockSpec((_BB, _M), lambda i: (jnp.where(i < _TA, i, i - _TA), 0)),
        out_shape=jax.ShapeDtypeStruct((_N, _M), jnp.float32),
        compiler_params=pltpu.CompilerParams(
            vmem_limit_bytes=100 * 1024 * 1024,
            internal_scratch_in_bytes=256 * 1024,
        ),
        scratch_shapes=[
            pltpu.VMEM((_N, _M), jnp.bfloat16),
            pltpu.VMEM((1, _M), jnp.float32),
            pltpu.VMEM((1, _M), jnp.float32),
        ],
    )(x, W, b2, g2, beta2)


# speedup vs baseline: 1.2186x; 1.2186x over previous
import jax
import jax.numpy as jnp
from jax.experimental import pallas as pl
from jax.experimental.pallas import tpu as pltpu

_N = 100000
_K = 384
_M = 128
_BA = 5000
_TA2 = _N // (2 * _BA)
_BB = 5000
_TB = _N // _BB


def _bn_linear_kernel(xa_ref, xb_ref, w_ref, b_ref, g_ref, beta_ref, out_ref,
                      y_ref, sum_ref, sq_ref):
    i = pl.program_id(0)

    @pl.when(i < _TA2)
    def _compute_phase():
        y1 = jax.lax.dot_general(
            xa_ref[...], w_ref[...],
            dimension_numbers=(((1,), (1,)), ((), ())),
            preferred_element_type=jnp.float32,
        ) + b_ref[...]
        y2 = jax.lax.dot_general(
            xb_ref[...], w_ref[...],
            dimension_numbers=(((1,), (1,)), ((), ())),
            preferred_element_type=jnp.float32,
        ) + b_ref[...]
        base = i * (2 * _BA)
        y_ref[pl.ds(base, _BA), :] = y1.astype(jnp.bfloat16)
        y_ref[pl.ds(base + _BA, _BA), :] = y2.astype(jnp.bfloat16)
        ps = jnp.sum(y1, axis=0, keepdims=True) + jnp.sum(y2, axis=0, keepdims=True)
        pq = jnp.sum(y1 * y1, axis=0, keepdims=True) + jnp.sum(y2 * y2, axis=0, keepdims=True)

        @pl.when(i == 0)
        def _():
            sum_ref[...] = ps
            sq_ref[...] = pq

        @pl.when(i > 0)
        def _():
            sum_ref[...] += ps
            sq_ref[...] += pq

    @pl.when(i >= _TA2)
    def _normalize_phase():
        j = i - _TA2
        inv_n = 1.0 / _N
        mean = sum_ref[...] * inv_n
        var = sq_ref[...] * inv_n - mean * mean
        scale = g_ref[...] * jax.lax.rsqrt(var + 1e-5)
        shift = beta_ref[...] - mean * scale
        out_ref[...] = y_ref[pl.ds(j * _BB, _BB), :].astype(jnp.float32) * scale + shift


@jax.jit
def kernel(x, W, b, gamma, beta):
    b2 = b.reshape(1, _M)
    g2 = gamma.reshape(1, _M)
    beta2 = beta.reshape(1, _M)
    return pl.pallas_call(
        _bn_linear_kernel,
        grid=(_TA2 + _TB,),
        in_specs=[
            pl.BlockSpec((_BA, _K),
                         lambda i: (jnp.minimum(2 * i, 2 * _TA2 - 2), 0)),
            pl.BlockSpec((_BA, _K),
                         lambda i: (jnp.minimum(2 * i + 1, 2 * _TA2 - 1), 0)),
            pl.BlockSpec((_M, _K), lambda i: (0, 0)),
            pl.BlockSpec((1, _M), lambda i: (0, 0)),
            pl.BlockSpec((1, _M), lambda i: (0, 0)),
            pl.BlockSpec((1, _M), lambda i: (0, 0)),
        ],
        out_specs=pl.BlockSpec((_BB, _M), lambda i: (jnp.maximum(i - _TA2, 0), 0)),
        out_shape=jax.ShapeDtypeStruct((_N, _M), jnp.float32),
        compiler_params=pltpu.CompilerParams(
            vmem_limit_bytes=100 * 1024 * 1024,
        ),
        scratch_shapes=[
            pltpu.VMEM((_N, _M), jnp.bfloat16),
            pltpu.VMEM((1, _M), jnp.float32),
            pltpu.VMEM((1, _M), jnp.float32),
        ],
    )(x, x, W, b2, g2, beta2)


# speedup vs baseline: 1.2315x; 1.0106x over previous
import jax
import jax.numpy as jnp
from jax.experimental import pallas as pl
from jax.experimental.pallas import tpu as pltpu

_N = 100000
_K = 384
_M = 128
_BA = 10000
_TA = _N // _BA
_BB = 5000
_TB = _N // _BB


def _bn_linear_kernel(x_ref, w_ref, b_ref, g_ref, beta_ref, out_ref,
                      y_ref, sum_ref, sq_ref):
    i = pl.program_id(0)

    @pl.when(i < _TA)
    def _compute_phase():
        y = jax.lax.dot_general(
            x_ref[...], w_ref[...],
            dimension_numbers=(((1,), (1,)), ((), ())),
            preferred_element_type=jnp.float32,
        ) + b_ref[...]
        y_ref[pl.ds(i * _BA, _BA), :] = y.astype(jnp.bfloat16)
        ps = jnp.sum(y, axis=0, keepdims=True)
        pq = jnp.sum(y * y, axis=0, keepdims=True)

        @pl.when(i == 0)
        def _():
            sum_ref[...] = ps
            sq_ref[...] = pq

        @pl.when(i > 0)
        def _():
            sum_ref[...] += ps
            sq_ref[...] += pq

    @pl.when(i >= _TA)
    def _normalize_phase():
        j = i - _TA
        inv_n = 1.0 / _N
        mean = sum_ref[...] * inv_n
        var = sq_ref[...] * inv_n - mean * mean
        scale = g_ref[...] * jax.lax.rsqrt(var + 1e-5)
        shift = beta_ref[...] - mean * scale
        out_ref[...] = y_ref[pl.ds(j * _BB, _BB), :].astype(jnp.float32) * scale + shift


@jax.jit
def kernel(x, W, b, gamma, beta):
    b2 = b.reshape(1, _M)
    g2 = gamma.reshape(1, _M)
    beta2 = beta.reshape(1, _M)
    return pl.pallas_call(
        _bn_linear_kernel,
        grid=(_TA + _TB,),
        in_specs=[
            pl.BlockSpec((_BA, _K), lambda i: (jnp.minimum(i, _TA - 1), 0)),
            pl.BlockSpec((_M, _K), lambda i: (0, 0)),
            pl.BlockSpec((1, _M), lambda i: (0, 0)),
            pl.BlockSpec((1, _M), lambda i: (0, 0)),
            pl.BlockSpec((1, _M), lambda i: (0, 0)),
        ],
        out_specs=pl.BlockSpec((_BB, _M), lambda i: (jnp.maximum(i - _TA, 0), 0)),
        out_shape=jax.ShapeDtypeStruct((_N, _M), jnp.float32),
        compiler_params=pltpu.CompilerParams(
            vmem_limit_bytes=100 * 1024 * 1024,
        ),
        scratch_shapes=[
            pltpu.VMEM((_N, _M), jnp.bfloat16),
            pltpu.VMEM((1, _M), jnp.float32),
            pltpu.VMEM((1, _M), jnp.float32),
        ],
    )(x, W, b2, g2, beta2)
